# baseline (speedup 1.0000x reference)
import functools

import jax
import jax.numpy as jnp
from jax import lax
from jax.experimental import pallas as pl
from jax.experimental.pallas import tpu as pltpu
from jax.experimental.pallas import tpu_sc as plsc

_H = 16
_EMB = 100000
_TP = 32
_B = 1024
_T2 = 512
_NW = 32
_RPW = _T2 // _NW


def _sc_gather_body(idx_hbm, table_hbm, out_hbm, idx_v, out_v, sem):
    wid = lax.axis_index("s") * 2 + lax.axis_index("c")
    base = wid * _RPW
    pltpu.sync_copy(idx_hbm.at[pl.ds(base, _RPW)], idx_v)
    iv = idx_v[...]
    copies = [
        pltpu.async_copy(
            table_hbm.at[jnp.int32(k)].at[iv], out_v.at[jnp.int32(k)], sem)
        for k in range(_TP)
    ]
    for cp in copies:
        cp.wait()
    pltpu.sync_copy(out_v, out_hbm.at[:, pl.ds(base, _RPW)])


def _sc_gather(idx2_i32, table_flat_i32):
    mesh = plsc.VectorSubcoreMesh(core_axis_name="c", subcore_axis_name="s")
    f = functools.partial(
        pl.kernel,
        mesh=mesh,
        compiler_params=pltpu.CompilerParams(use_tc_tiling_on_sc=False),
        out_type=jax.ShapeDtypeStruct((_TP, _T2), jnp.int32),
        scratch_types=[
            pltpu.VMEM((_RPW,), jnp.int32),
            pltpu.VMEM((_TP, _RPW), jnp.int32),
            pltpu.SemaphoreType.DMA,
        ],
    )(_sc_gather_body)
    return f(idx2_i32, table_flat_i32)


def _tc_body(i1c_ref, i1r_ref, i2r_ref, cand_ref, candt_ref, gt_ref,
             norm_ref, out_ref):
    i1c = i1c_ref[:, :]
    i1r = i1r_ref[:, :]
    i2r = i2r_ref[:, :]
    cand = cand_ref[:, :]
    candt = candt_ref[:, :]

    jrow = lax.broadcasted_iota(jnp.int32, (_B, _B), 1)
    eq = i1c == i1r
    wj = jnp.max(jnp.where(eq, jrow, jnp.int32(-1)), axis=1, keepdims=True)
    onehot = jnp.logical_and(eq, jrow == wj).astype(jnp.float32)
    hi = (cand >> 8).astype(jnp.float32)
    lo = (cand & 255).astype(jnp.float32)
    dot = functools.partial(
        lax.dot_general,
        dimension_numbers=(((1,), (0,)), ((), ())),
        preferred_element_type=jnp.float32,
    )
    sta = (dot(onehot, hi) * 256.0 + dot(onehot, lo)).astype(jnp.int32)

    jcol = lax.broadcasted_iota(jnp.int32, (_B, _T2), 0)
    eq2 = i1c == i2r
    w2 = jnp.max(jnp.where(eq2, jcol, jnp.int32(-1)), axis=0, keepdims=True)
    onehot2 = jnp.logical_and(eq2, jcol == w2).astype(jnp.float32)
    hit = (candt >> 8).astype(jnp.float32)
    lot = (candt & 255).astype(jnp.float32)
    pos_c = (dot(hit, onehot2) * 256.0 + dot(lot, onehot2)).astype(jnp.int32)
    has = w2 >= 0
    gt = gt_ref[:, :]

    acc = jnp.zeros((_B, _T2), jnp.int32)
    s_sum = jnp.zeros((1, _T2), jnp.int32)
    for k in range(_TP):
        pos_ck = lax.slice(pos_c, (k, 0), (k + 1, _T2))
        g_k = lax.slice(gt, (k, 0), (k + 1, _T2))
        pos_k = jnp.where(has, pos_ck, g_k)
        sb_k = jnp.where(pos_k >= 0, jnp.int32(1), jnp.int32(-1))
        pb_k = jnp.abs(pos_k)
        s_sum = s_sum + sb_k
        sa_k = lax.slice(sta, (0, k), (_B, k + 1))
        xorv = lax.bitwise_xor(sa_k, pb_k)
        fp = (xorv + 1).astype(jnp.float32)
        ebias = lax.shift_right_logical(
            lax.bitcast_convert_type(fp, jnp.int32), jnp.int32(23))
        acc = acc + sb_k * ebias
    m = (jnp.int32(142) * s_sum - acc).astype(jnp.float32) * 0.0625
    out_ref[:, :] = norm_ref[:, :] * m


def _tc_main(idx1_i32, idx2_i32, cand_i32, gathered_t, norm):
    return pl.pallas_call(
        _tc_body,
        out_shape=jax.ShapeDtypeStruct((_B, _T2), jnp.float32),
    )(
        idx1_i32.reshape(_B, 1),
        idx1_i32.reshape(1, _B),
        idx2_i32.reshape(1, _T2),
        cand_i32,
        cand_i32.T,
        gathered_t,
        norm,
    )


def kernel(idx1, idx2, norm, cand_loc, locations):
    idx1_i = idx1.astype(jnp.int32)
    idx2_i = idx2.astype(jnp.int32)
    cand_i = cand_loc.astype(jnp.int32)
    table_t = locations.T.astype(jnp.int32)
    gathered_t = _sc_gather(idx2_i, table_t)
    return _tc_main(idx1_i, idx2_i, cand_i, gathered_t, norm)


# speedup vs baseline: 1.0132x; 1.0132x over previous
import functools

import jax
import jax.numpy as jnp
from jax import lax
from jax.experimental import pallas as pl
from jax.experimental.pallas import tpu as pltpu
from jax.experimental.pallas import tpu_sc as plsc

_H = 16
_EMB = 100000
_TP = 32
_B = 1024
_T2 = 512
_NW = 32
_RPW = _T2 // _NW


def _sc_gather_body(idx_hbm, table_hbm, out_hbm, idx_v, out_v, sem):
    wid = lax.axis_index("s") * 2 + lax.axis_index("c")
    base = wid * _RPW
    pltpu.sync_copy(idx_hbm.at[pl.ds(base, _RPW)], idx_v)
    iv = idx_v[...]
    copies = [
        pltpu.async_copy(
            table_hbm.at[jnp.int32(k)].at[iv], out_v.at[jnp.int32(k)], sem)
        for k in range(_TP)
    ]
    for cp in copies:
        cp.wait()
    pltpu.sync_copy(out_v, out_hbm.at[:, pl.ds(base, _RPW)])


def _sc_gather(idx2_i32, table_flat_i32):
    mesh = plsc.VectorSubcoreMesh(core_axis_name="c", subcore_axis_name="s")
    f = functools.partial(
        pl.kernel,
        mesh=mesh,
        compiler_params=pltpu.CompilerParams(use_tc_tiling_on_sc=False),
        out_type=jax.ShapeDtypeStruct((_TP, _T2), jnp.int32),
        scratch_types=[
            pltpu.VMEM((_RPW,), jnp.int32),
            pltpu.VMEM((_TP, _RPW), jnp.int32),
            pltpu.SemaphoreType.DMA,
        ],
    )(_sc_gather_body)
    return f(idx2_i32, table_flat_i32)


def _tc_body(i1c_ref, i1r_ref, i2r_ref, cand_ref, candt_ref, gt_ref,
             norm_ref, out_ref):
    i1c = i1c_ref[:, :]
    i1r = i1r_ref[:, :]
    i2r = i2r_ref[:, :]
    cand = cand_ref[:, :]
    candt = candt_ref[:, :]

    jrow = lax.broadcasted_iota(jnp.int32, (_B, _B), 1)
    eq = i1c == i1r
    wj = jnp.max(jnp.where(eq, jrow, jnp.int32(-1)), axis=1, keepdims=True)
    onehot = jnp.logical_and(eq, jrow == wj).astype(jnp.float32)
    hi = (cand >> 8).astype(jnp.float32)
    lo = (cand & 255).astype(jnp.float32)
    dot = functools.partial(
        lax.dot_general,
        dimension_numbers=(((1,), (0,)), ((), ())),
        preferred_element_type=jnp.float32,
    )
    sta = (dot(onehot, hi) * 256.0 + dot(onehot, lo)).astype(jnp.int32)

    jcol = lax.broadcasted_iota(jnp.int32, (_B, _T2), 0)
    eq2 = i1c == i2r
    w2 = jnp.max(jnp.where(eq2, jcol, jnp.int32(-1)), axis=0, keepdims=True)
    onehot2 = jnp.logical_and(eq2, jcol == w2).astype(jnp.float32)
    hit = (candt >> 8).astype(jnp.float32)
    lot = (candt & 255).astype(jnp.float32)
    pos_c = (dot(hit, onehot2) * 256.0 + dot(lot, onehot2)).astype(jnp.int32)
    has = w2 >= 0
    gt = gt_ref[:, :]

    acc = jnp.zeros((_B, _T2), jnp.int32)
    s_sum = jnp.zeros((1, _T2), jnp.int32)
    for k in range(_TP):
        pos_ck = lax.slice(pos_c, (k, 0), (k + 1, _T2))
        g_k = lax.slice(gt, (k, 0), (k + 1, _T2))
        pos_k = jnp.where(has, pos_ck, g_k)
        sb_k = jnp.where(pos_k >= 0, jnp.int32(1), jnp.int32(-1))
        pb_k = jnp.abs(pos_k)
        s_sum = s_sum + sb_k
        sa_k = lax.slice(sta, (0, k), (_B, k + 1))
        xorv = lax.bitwise_xor(sa_k, pb_k)
        acc = acc + sb_k * lax.clz(xorv + 1)
    m = (acc - jnp.int32(16) * s_sum).astype(jnp.float32) * 0.0625
    out_ref[:, :] = norm_ref[:, :] * m


def _tc_main(idx1_i32, idx2_i32, cand_i32, gathered_t, norm):
    return pl.pallas_call(
        _tc_body,
        out_shape=jax.ShapeDtypeStruct((_B, _T2), jnp.float32),
    )(
        idx1_i32.reshape(_B, 1),
        idx1_i32.reshape(1, _B),
        idx2_i32.reshape(1, _T2),
        cand_i32,
        cand_i32.T,
        gathered_t,
        norm,
    )


def kernel(idx1, idx2, norm, cand_loc, locations):
    idx1_i = idx1.astype(jnp.int32)
    idx2_i = idx2.astype(jnp.int32)
    cand_i = cand_loc.astype(jnp.int32)
    table_t = locations.T.astype(jnp.int32)
    gathered_t = _sc_gather(idx2_i, table_t)
    return _tc_main(idx1_i, idx2_i, cand_i, gathered_t, norm)
